# baseline (speedup 1.0000x reference)
import functools

import jax
import jax.numpy as jnp
from jax import lax
from jax.experimental import pallas as pl
from jax.experimental.pallas import tpu as pltpu
from jax.experimental.pallas import tpu_sc as plsc

N = 10000
E = 320000
D = 128

NC = 2
NS = 16
NW = NC * NS
EPW = E // NW
C = 80
STEPS = EPW // C
NBUF = 2
RPT = 624
LAST = N - (NS - 1) * RPT
NLANE = D // 16


def _sc_propagate(ei4, norm_flat, rel, x, target):
    mesh = plsc.VectorSubcoreMesh(core_axis_name="c", subcore_axis_name="s")

    scratch = (
        [pltpu.VMEM((1, C), jnp.int32) for _ in range(NBUF)]
        + [pltpu.VMEM((1, C), jnp.int32) for _ in range(NBUF)]
        + [pltpu.VMEM((C,), jnp.float32) for _ in range(NBUF)]
        + [pltpu.VMEM((C, D), jnp.float32) for _ in range(NBUF)]
        + [pltpu.VMEM((C, D), jnp.float32) for _ in range(NBUF)]
        + [pltpu.VMEM_SHARED((N, D), jnp.float32)]
        + [pltpu.SemaphoreType.DMA for _ in range(6 * NBUF)]
    )

    @functools.partial(
        pl.kernel,
        out_type=jax.ShapeDtypeStruct((NC, N, D), jnp.float32),
        mesh=mesh,
        scratch_types=scratch,
    )
    def k(ei_hbm, norm_hbm, rel_hbm, x_hbm, tgt_hbm, out_hbm, *sc):
        srcb = list(sc[0:NBUF])
        dsth = list(sc[NBUF:2 * NBUF])
        normb = list(sc[2 * NBUF:3 * NBUF])
        relb = list(sc[3 * NBUF:4 * NBUF])
        xrb = list(sc[4 * NBUF:5 * NBUF])
        h_sh = sc[5 * NBUF]
        sems = sc[5 * NBUF + 1:]
        sem_s = list(sems[0:NBUF])
        sem_d = list(sems[NBUF:2 * NBUF])
        sem_n = list(sems[2 * NBUF:3 * NBUF])
        sem_r = list(sems[3 * NBUF:4 * NBUF])
        sem_g = list(sems[4 * NBUF:5 * NBUF])
        sem_sc = list(sems[5 * NBUF:6 * NBUF])

        cid = lax.axis_index("c")
        sid = lax.axis_index("s")
        wid = sid * NC + cid
        base_e = wid * EPW

        def fire_src(b, kk):
            pltpu.async_copy(ei_hbm.at[0, wid, kk], srcb[b], sem_s[b])

        def wait_src(b, kk):
            pltpu.make_async_copy(ei_hbm.at[0, wid, kk], srcb[b],
                                  sem_s[b]).wait()

        def fire_dst(b, kk):
            pltpu.async_copy(ei_hbm.at[1, wid, kk], dsth[b], sem_d[b])

        def wait_dst(b, kk):
            pltpu.make_async_copy(ei_hbm.at[1, wid, kk], dsth[b],
                                  sem_d[b]).wait()

        def fire_norm(b, kk):
            pltpu.async_copy(norm_hbm.at[pl.ds(base_e + kk * C, C)], normb[b],
                             sem_n[b])

        def fire_rel(b, kk):
            pltpu.async_copy(rel_hbm.at[pl.ds(base_e + kk * C, C)], relb[b],
                             sem_r[b])

        def fire_gather(b):
            pltpu.async_copy(x_hbm.at[srcb[b].at[0]], xrb[b], sem_g[b])

        def wait_scatter(b):
            pltpu.make_async_copy(relb[b], h_sh.at[dsth[b].at[0]],
                                  sem_sc[b]).wait()

        def chunk_step(b, kk, tail):
            bo = 1 - b
            pltpu.make_async_copy(norm_hbm.at[pl.ds(base_e + kk * C, C)],
                                  normb[b], sem_n[b]).wait()
            pltpu.make_async_copy(rel_hbm.at[pl.ds(base_e + kk * C, C)],
                                  relb[b], sem_r[b]).wait()

            if tail:
                wait_scatter(bo)
            else:
                @pl.when(kk >= 1)
                def _():
                    wait_scatter(bo)

                @pl.when(kk + 1 < STEPS)
                def _():
                    fire_rel(bo, kk + 1)
                    fire_dst(bo, kk + 1)

            pltpu.make_async_copy(x_hbm.at[srcb[b].at[0]], xrb[b],
                                  sem_g[b]).wait()

            if not tail:
                @pl.when(kk + NBUF < STEPS)
                def _():
                    fire_src(b, kk + NBUF)

            rel_v = relb[b]
            xr_v = xrb[b]
            norm_v = normb[b]

            @plsc.parallel_loop(0, C, step=8)
            def _(e0):
                nvv = norm_v[pl.ds(e0, 16)]
                for i in range(8):
                    e = e0 + i
                    nv = nvv[i]
                    for j in range(NLANE):
                        sl = pl.ds(j * 16, 16)
                        rel_v[e, sl] = xr_v[e, sl] * rel_v[e, sl] * nv

            wait_dst(b, kk)
            pltpu.async_copy(rel_v, h_sh.at[dsth[b].at[0]], sem_sc[b],
                             add=True)

            if not tail:
                @pl.when(kk + NBUF < STEPS)
                def _():
                    fire_norm(b, kk + NBUF)
                    wait_src(b, kk + NBUF)
                    fire_gather(b)

        r0 = sid * RPT

        @pl.when(cid == 0)
        def _():
            @pl.when(sid < NS - 1)
            def _():
                pltpu.sync_copy(tgt_hbm.at[pl.ds(r0, RPT)],
                                h_sh.at[pl.ds(r0, RPT)])

            @pl.when(sid == NS - 1)
            def _():
                pltpu.sync_copy(tgt_hbm.at[pl.ds((NS - 1) * RPT, LAST)],
                                h_sh.at[pl.ds((NS - 1) * RPT, LAST)])

        @pl.when(cid == 1)
        def _():
            z_v = xrb[0]

            @pl.loop(0, C)
            def _(e):
                for j in range(NLANE):
                    z_v[e, pl.ds(j * 16, 16)] = jnp.zeros((16,), jnp.float32)

            for m in range(7):
                pltpu.sync_copy(z_v, h_sh.at[pl.ds(r0 + m * C, C)])

            @pl.when(sid < NS - 1)
            def _():
                pltpu.sync_copy(z_v.at[pl.ds(0, RPT - 7 * C)],
                                h_sh.at[pl.ds(r0 + 7 * C, RPT - 7 * C)])

            @pl.when(sid == NS - 1)
            def _():
                pltpu.sync_copy(z_v, h_sh.at[pl.ds(r0 + 7 * C, C)])

        plsc.subcore_barrier()

        for b in range(NBUF):
            fire_src(b, b)
            fire_norm(b, b)
        fire_rel(0, 0)
        fire_dst(0, 0)
        for b in range(NBUF):
            wait_src(b, b)
            fire_gather(b)

        @pl.loop(0, STEPS - 1, step=NBUF)
        def _(k0):
            for b in range(NBUF):
                chunk_step(b, k0 + b, False)

        chunk_step(0, STEPS - 1, True)
        wait_scatter(0)

        plsc.subcore_barrier()

        @pl.when(sid < NS - 1)
        def _():
            pltpu.sync_copy(h_sh.at[pl.ds(r0, RPT)],
                            out_hbm.at[cid, pl.ds(r0, RPT)])

        @pl.when(sid == NS - 1)
        def _():
            pltpu.sync_copy(h_sh.at[pl.ds((NS - 1) * RPT, LAST)],
                            out_hbm.at[cid, pl.ds((NS - 1) * RPT, LAST)])

    return k(ei4, norm_flat, rel, x, target)


BR = 400


def _tc_body(hp_ref, w_ref, b_ref, o_ref):
    h = hp_ref[0] + hp_ref[1]
    acc = lax.dot_general(h, w_ref[...], (((1,), (1,)), ((), ())),
                          preferred_element_type=jnp.float32)
    o_ref[...] = jnp.maximum(acc + b_ref[...], 0.0)


def _tc_linear(hp, w, b2):
    return pl.pallas_call(
        _tc_body,
        grid=(N // BR,),
        in_specs=[
            pl.BlockSpec((NC, BR, D), lambda i: (0, i, 0)),
            pl.BlockSpec((D, D), lambda i: (0, 0)),
            pl.BlockSpec((1, D), lambda i: (0, 0)),
        ],
        out_specs=pl.BlockSpec((BR, D), lambda i: (i, 0)),
        out_shape=jax.ShapeDtypeStruct((N, D), jnp.float32),
    )(hp, w, b2)


def kernel(x, edge_index, norm, edge_rel_emd, target_rel_emd_new, W_line,
           b_line):
    ei4 = edge_index.astype(jnp.int32).reshape(2, NW, STEPS, 1, C)
    hp = _sc_propagate(ei4, norm.reshape(E), edge_rel_emd, x,
                       target_rel_emd_new)
    return _tc_linear(hp, W_line, b_line.reshape(1, D))


# speedup vs baseline: 1.3634x; 1.3634x over previous
import functools

import jax
import jax.numpy as jnp
from jax import lax
from jax.experimental import pallas as pl
from jax.experimental.pallas import tpu as pltpu
from jax.experimental.pallas import tpu_sc as plsc

N = 10000
E = 320000
D = 128

NC = 2
NS = 16
NW = NC * NS
EPW = E // NW
C = 80
STEPS = EPW // C
NBUF = 2
RPT = 624
LAST = N - (NS - 1) * RPT
NLANE = D // 16


def _sc_propagate(ei4, norm_flat, rel, x, target):
    mesh = plsc.VectorSubcoreMesh(core_axis_name="c", subcore_axis_name="s")

    scratch = (
        [pltpu.VMEM((1, C), jnp.int32) for _ in range(NBUF)]
        + [pltpu.VMEM((1, C), jnp.int32) for _ in range(NBUF)]
        + [pltpu.VMEM((C,), jnp.float32) for _ in range(NBUF)]
        + [pltpu.VMEM((C, D), jnp.float32) for _ in range(NBUF)]
        + [pltpu.VMEM((C, D), jnp.float32) for _ in range(NBUF)]
        + [pltpu.VMEM_SHARED((N, D), jnp.float32)]
        + [pltpu.SemaphoreType.DMA for _ in range(6 * NBUF)]
    )

    @functools.partial(
        pl.kernel,
        out_type=jax.ShapeDtypeStruct((NC, N, D), jnp.float32),
        mesh=mesh,
        scratch_types=scratch,
    )
    def k(ei_hbm, norm_hbm, rel_hbm, x_hbm, tgt_hbm, out_hbm, *sc):
        srcb = list(sc[0:NBUF])
        dsth = list(sc[NBUF:2 * NBUF])
        normb = list(sc[2 * NBUF:3 * NBUF])
        relb = list(sc[3 * NBUF:4 * NBUF])
        xrb = list(sc[4 * NBUF:5 * NBUF])
        h_sh = sc[5 * NBUF]
        sems = sc[5 * NBUF + 1:]
        sem_s = list(sems[0:NBUF])
        sem_d = list(sems[NBUF:2 * NBUF])
        sem_n = list(sems[2 * NBUF:3 * NBUF])
        sem_r = list(sems[3 * NBUF:4 * NBUF])
        sem_g = list(sems[4 * NBUF:5 * NBUF])
        sem_sc = list(sems[5 * NBUF:6 * NBUF])

        cid = lax.axis_index("c")
        sid = lax.axis_index("s")
        wid = sid * NC + cid
        base_e = wid * EPW

        def fire_src(b, kk):
            pltpu.async_copy(ei_hbm.at[0, wid, kk], srcb[b], sem_s[b])

        def wait_src(b, kk):
            pltpu.make_async_copy(ei_hbm.at[0, wid, kk], srcb[b],
                                  sem_s[b]).wait()

        def fire_dst(b, kk):
            pltpu.async_copy(ei_hbm.at[1, wid, kk], dsth[b], sem_d[b])

        def wait_dst(b, kk):
            pltpu.make_async_copy(ei_hbm.at[1, wid, kk], dsth[b],
                                  sem_d[b]).wait()

        def fire_norm(b, kk):
            pltpu.async_copy(norm_hbm.at[pl.ds(base_e + kk * C, C)], normb[b],
                             sem_n[b])

        def fire_rel(b, kk):
            pltpu.async_copy(rel_hbm.at[pl.ds(base_e + kk * C, C)], relb[b],
                             sem_r[b])

        def fire_gather(b):
            pltpu.async_copy(x_hbm.at[srcb[b].at[0]], xrb[b], sem_g[b])

        def wait_scatter(b):
            pltpu.make_async_copy(relb[b], h_sh.at[dsth[b].at[0]],
                                  sem_sc[b]).wait()

        def chunk_step(b, kk, tail):
            bo = 1 - b
            pltpu.make_async_copy(norm_hbm.at[pl.ds(base_e + kk * C, C)],
                                  normb[b], sem_n[b]).wait()
            pltpu.make_async_copy(rel_hbm.at[pl.ds(base_e + kk * C, C)],
                                  relb[b], sem_r[b]).wait()

            if tail:
                wait_scatter(bo)
            else:
                @pl.when(kk >= 1)
                def _():
                    wait_scatter(bo)

                @pl.when(kk + 1 < STEPS)
                def _():
                    fire_rel(bo, kk + 1)
                    fire_dst(bo, kk + 1)

            pltpu.make_async_copy(x_hbm.at[srcb[b].at[0]], xrb[b],
                                  sem_g[b]).wait()

            if not tail:
                @pl.when(kk + NBUF < STEPS)
                def _():
                    fire_src(b, kk + NBUF)

            rel_v = relb[b]
            xr_v = xrb[b]
            norm_v = normb[b]

            @plsc.parallel_loop(0, C, step=16)
            def _(e0):
                nvv = norm_v[pl.ds(e0, 16)]
                for i in range(16):
                    e = e0 + i
                    nv = nvv[i]
                    for j in range(NLANE):
                        sl = pl.ds(j * 16, 16)
                        rel_v[e, sl] = xr_v[e, sl] * rel_v[e, sl] * nv

            wait_dst(b, kk)
            pltpu.async_copy(rel_v, h_sh.at[dsth[b].at[0]], sem_sc[b],
                             add=True)

            if not tail:
                @pl.when(kk + NBUF < STEPS)
                def _():
                    fire_norm(b, kk + NBUF)
                    wait_src(b, kk + NBUF)
                    fire_gather(b)

        r0 = sid * RPT

        @pl.when(cid == 0)
        def _():
            @pl.when(sid < NS - 1)
            def _():
                pltpu.sync_copy(tgt_hbm.at[pl.ds(r0, RPT)],
                                h_sh.at[pl.ds(r0, RPT)])

            @pl.when(sid == NS - 1)
            def _():
                pltpu.sync_copy(tgt_hbm.at[pl.ds((NS - 1) * RPT, LAST)],
                                h_sh.at[pl.ds((NS - 1) * RPT, LAST)])

        @pl.when(cid == 1)
        def _():
            z_v = xrb[0]

            @pl.loop(0, C)
            def _(e):
                for j in range(NLANE):
                    z_v[e, pl.ds(j * 16, 16)] = jnp.zeros((16,), jnp.float32)

            for m in range(7):
                pltpu.sync_copy(z_v, h_sh.at[pl.ds(r0 + m * C, C)])

            @pl.when(sid < NS - 1)
            def _():
                pltpu.sync_copy(z_v.at[pl.ds(0, RPT - 7 * C)],
                                h_sh.at[pl.ds(r0 + 7 * C, RPT - 7 * C)])

            @pl.when(sid == NS - 1)
            def _():
                pltpu.sync_copy(z_v, h_sh.at[pl.ds(r0 + 7 * C, C)])

        plsc.subcore_barrier()

        for b in range(NBUF):
            fire_src(b, b)
            fire_norm(b, b)
        fire_rel(0, 0)
        fire_dst(0, 0)
        for b in range(NBUF):
            wait_src(b, b)
            fire_gather(b)

        @pl.loop(0, STEPS - 1, step=NBUF)
        def _(k0):
            for b in range(NBUF):
                chunk_step(b, k0 + b, False)

        chunk_step(0, STEPS - 1, True)
        wait_scatter(0)

        plsc.subcore_barrier()

        @pl.when(sid < NS - 1)
        def _():
            pltpu.sync_copy(h_sh.at[pl.ds(r0, RPT)],
                            out_hbm.at[cid, pl.ds(r0, RPT)])

        @pl.when(sid == NS - 1)
        def _():
            pltpu.sync_copy(h_sh.at[pl.ds((NS - 1) * RPT, LAST)],
                            out_hbm.at[cid, pl.ds((NS - 1) * RPT, LAST)])

    return k(ei4, norm_flat, rel, x, target)


BR = 400


def _tc_body(hp_ref, w_ref, b_ref, o_ref):
    h = hp_ref[0] + hp_ref[1]
    acc = lax.dot_general(h, w_ref[...], (((1,), (1,)), ((), ())),
                          preferred_element_type=jnp.float32)
    o_ref[...] = jnp.maximum(acc + b_ref[...], 0.0)


def _tc_linear(hp, w, b2):
    return pl.pallas_call(
        _tc_body,
        grid=(N // BR,),
        in_specs=[
            pl.BlockSpec((NC, BR, D), lambda i: (0, i, 0)),
            pl.BlockSpec((D, D), lambda i: (0, 0)),
            pl.BlockSpec((1, D), lambda i: (0, 0)),
        ],
        out_specs=pl.BlockSpec((BR, D), lambda i: (i, 0)),
        out_shape=jax.ShapeDtypeStruct((N, D), jnp.float32),
    )(hp, w, b2)


def kernel(x, edge_index, norm, edge_rel_emd, target_rel_emd_new, W_line,
           b_line):
    ei4 = edge_index.astype(jnp.int32).reshape(2, NW, STEPS, 1, C)
    hp = _sc_propagate(ei4, norm.reshape(E), edge_rel_emd, x,
                       target_rel_emd_new)
    return _tc_linear(hp, W_line, b_line.reshape(1, D))
